# baseline (speedup 1.0000x reference)
import functools

import numpy as np
import jax
import jax.numpy as jnp
from jax import lax
from jax.experimental import pallas as pl
from jax.experimental.pallas import tpu as pltpu
from jax.experimental.pallas import tpu_sc as plsc

_B, _T, _D, _V = 1024, 200, 128, 100000
_SCALE = float(_D) ** 0.5
_NC, _NS = 2, 16
_NW = _NC * _NS
_SPW = _B // _NW
_IDX_MINOR = 100
_NBUF = 3
_LOOK = 2
_NLOOP = (_SPW - _LOOK) // _NBUF


def _pos_table():
    pos = np.arange(_T, dtype=np.float32)[:, None]
    div = np.exp(
        np.arange(0, _D, 2, dtype=np.float32) * (-np.log(10000.0) / _D)
    ).astype(np.float32)
    pe = np.zeros((_T, _D), dtype=np.float32)
    pe[:, 0::2] = np.sin(pos * div)
    pe[:, 1::2] = np.cos(pos * div)
    return pe


def _sc_body(idx_hbm, table_hbm, pos_hbm, out_hbm, pos_v, idx_v, bufs,
             gsems, ssems):
    wid = lax.axis_index("s") * _NC + lax.axis_index("c")
    seq0 = wid * _SPW

    pltpu.sync_copy(pos_hbm, pos_v)
    pltpu.sync_copy(idx_hbm.at[pl.ds(wid * (_SPW * 2), _SPW * 2)], idx_v)

    def gather_start(i, slot):
        for h in range(2):
            pltpu.async_copy(
                table_hbm.at[idx_v.at[2 * i + h]],
                bufs[slot].at[pl.ds(h * _IDX_MINOR, _IDX_MINOR)],
                gsems[slot])

    def gather_wait(slot):
        for h in range(2):
            pltpu.make_async_copy(
                table_hbm.at[idx_v.at[h]],
                bufs[slot].at[pl.ds(h * _IDX_MINOR, _IDX_MINOR)],
                gsems[slot]).wait()

    def scatter_start(i, slot):
        pltpu.async_copy(
            bufs[slot], out_hbm.at[pl.ds((seq0 + i) * _T, _T)], ssems[slot])

    def scatter_wait(slot):
        pltpu.make_async_copy(
            bufs[slot], out_hbm.at[pl.ds(0, _T)], ssems[slot]).wait()

    def fma(slot):
        @plsc.parallel_loop(0, _T)
        def _row(r):
            for j in range(_D // 16):
                cs = pl.ds(j * 16, 16)
                bufs[slot][r, cs] = bufs[slot][r, cs] * _SCALE + pos_v[r, cs]

    def step(i, slot, nslot, in_loop):
        nxt = i + _LOOK
        if in_loop:
            @pl.when(nxt >= _NBUF)
            def _():
                scatter_wait(nslot)
            gather_start(nxt, nslot)
        gather_wait(slot)
        fma(slot)
        scatter_start(i, slot)

    for i in range(_LOOK):
        gather_start(i, i)

    def outer(g, carry):
        for b in range(_NBUF):
            i = g * _NBUF + b
            step(i, b, (b + _LOOK) % _NBUF, True)
        return carry

    lax.fori_loop(0, _NLOOP, outer, 0)

    for i in range(_NLOOP * _NBUF, _SPW):
        step(i, i % _NBUF, None, False)
    for k in range(_SPW - _NBUF, _SPW):
        scatter_wait(k % _NBUF)


@functools.partial(
    pl.kernel,
    out_type=jax.ShapeDtypeStruct((_B * _T, _D), jnp.float32),
    mesh=plsc.VectorSubcoreMesh(core_axis_name="c", subcore_axis_name="s"),
    scratch_types=[
        pltpu.VMEM((_T, _D), jnp.float32),
        pltpu.VMEM((_SPW * 2, _IDX_MINOR), jnp.int32),
        pltpu.VMEM((_T, _D), jnp.float32),
        pltpu.VMEM((_T, _D), jnp.float32),
        pltpu.VMEM((_T, _D), jnp.float32),
        pltpu.SemaphoreType.DMA,
        pltpu.SemaphoreType.DMA,
        pltpu.SemaphoreType.DMA,
        pltpu.SemaphoreType.DMA,
        pltpu.SemaphoreType.DMA,
        pltpu.SemaphoreType.DMA,
    ],
)
def _sc_embed(idx_hbm, table_hbm, pos_hbm, out_hbm, pos_v, idx_v,
              b0, b1, b2, g0, g1, g2, s0, s1, s2):
    _sc_body(idx_hbm, table_hbm, pos_hbm, out_hbm, pos_v, idx_v,
             [b0, b1, b2], [g0, g1, g2], [s0, s1, s2])


def kernel(input, table):
    idx = input.reshape(_B * _T).astype(jnp.int32).reshape(-1, _IDX_MINOR)
    pos = jnp.asarray(_pos_table())
    out = _sc_embed(idx, table, pos)
    return out.reshape(_B, _T, _D)


# speedup vs baseline: 1.1137x; 1.1137x over previous
import functools

import numpy as np
import jax
import jax.numpy as jnp
from jax import lax
from jax.experimental import pallas as pl
from jax.experimental.pallas import tpu as pltpu
from jax.experimental.pallas import tpu_sc as plsc

_B, _T, _D, _V = 1024, 200, 128, 100000
_SCALE = float(_D) ** 0.5
_NC, _NS = 2, 16
_NW = _NC * _NS
_SPW = _B // _NW
_IDX_MINOR = 100
_NBUF = 3
_RPI = 2


def _pos_table():
    pos = np.arange(_T, dtype=np.float32)[:, None]
    div = np.exp(
        np.arange(0, _D, 2, dtype=np.float32) * (-np.log(10000.0) / _D)
    ).astype(np.float32)
    pe = np.zeros((_T, _D), dtype=np.float32)
    pe[:, 0::2] = np.sin(pos * div)
    pe[:, 1::2] = np.cos(pos * div)
    return pe


def _sc_body(idx_hbm, table_hbm, pos_hbm, out_hbm, pos_v, idx_v, bufs,
             g0, g1, g2, s0, s1, s2):
    gsems = [g0, g1, g2]
    ssems = [s0, s1, s2]
    wid = lax.axis_index("s") * _NC + lax.axis_index("c")
    idx_row0 = wid * (_SPW * 2)
    seq0 = wid * _SPW

    pltpu.sync_copy(pos_hbm, pos_v)
    pltpu.sync_copy(idx_hbm.at[pl.ds(idx_row0, _SPW * 2)], idx_v)

    def gather_start(i, slot):
        cps = []
        for h in range(2):
            cps.append(pltpu.async_copy(
                table_hbm.at[idx_v.at[2 * i + h]],
                bufs.at[slot].at[pl.ds(h * _IDX_MINOR, _IDX_MINOR)],
                gsems[slot]))
        return cps

    def scatter_start(i, slot):
        return pltpu.async_copy(
            bufs.at[slot],
            out_hbm.at[pl.ds((seq0 + i) * _T, _T)],
            ssems[slot])

    def fma(slot):
        def body(rr, carry):
            for u in range(_RPI):
                r = rr * _RPI + u
                for j in range(_D // 16):
                    cs = pl.ds(j * 16, 16)
                    bufs[slot, r, cs] = (
                        bufs[slot, r, cs] * _SCALE + pos_v[r, cs])
            return carry
        lax.fori_loop(0, _T // _RPI, body, 0)

    pending_g = {0: gather_start(0, 0)}
    pending_s = {}
    for i in range(_SPW):
        slot = i % _NBUF
        if i + 1 < _SPW:
            nxt = (i + 1) % _NBUF
            if (i + 1) >= _NBUF:
                pending_s.pop(i + 1 - _NBUF).wait()
            pending_g[i + 1] = gather_start(i + 1, nxt)
        for cp in pending_g.pop(i):
            cp.wait()
        fma(slot)
        pending_s[i] = scatter_start(i, slot)
    for i in sorted(pending_s):
        pending_s.pop(i).wait()


@functools.partial(
    pl.kernel,
    out_type=jax.ShapeDtypeStruct((_B * _T, _D), jnp.float32),
    mesh=plsc.VectorSubcoreMesh(core_axis_name="c", subcore_axis_name="s"),
    scratch_types=[
        pltpu.VMEM((_T, _D), jnp.float32),
        pltpu.VMEM((_SPW * 2, _IDX_MINOR), jnp.int32),
        pltpu.VMEM((_NBUF, _T, _D), jnp.float32),
        pltpu.SemaphoreType.DMA,
        pltpu.SemaphoreType.DMA,
        pltpu.SemaphoreType.DMA,
        pltpu.SemaphoreType.DMA,
        pltpu.SemaphoreType.DMA,
        pltpu.SemaphoreType.DMA,
    ],
)
def _sc_embed(idx_hbm, table_hbm, pos_hbm, out_hbm, *rest):
    _sc_body(idx_hbm, table_hbm, pos_hbm, out_hbm, *rest)


def kernel(input, table):
    idx = input.reshape(_B * _T).astype(jnp.int32).reshape(-1, _IDX_MINOR)
    pos = jnp.asarray(_pos_table())
    out = _sc_embed(idx, table, pos)
    return out.reshape(_B, _T, _D)


# speedup vs baseline: 1.1518x; 1.0342x over previous
import functools

import numpy as np
import jax
import jax.numpy as jnp
from jax import lax
from jax.experimental import pallas as pl
from jax.experimental.pallas import tpu as pltpu
from jax.experimental.pallas import tpu_sc as plsc

_B, _T, _D, _V = 1024, 200, 128, 100000
_SCALE = float(_D) ** 0.5
_NC, _NS = 2, 16
_NW = _NC * _NS
_SPW = _B // _NW
_IDX_MINOR = 100
_NBUF = 3


def _pos_table():
    pos = np.arange(_T, dtype=np.float32)[:, None]
    div = np.exp(
        np.arange(0, _D, 2, dtype=np.float32) * (-np.log(10000.0) / _D)
    ).astype(np.float32)
    pe = np.zeros((_T, _D), dtype=np.float32)
    pe[:, 0::2] = np.sin(pos * div)
    pe[:, 1::2] = np.cos(pos * div)
    return pe


def _sc_body(idx_hbm, table_hbm, pos_hbm, out_hbm, pos_v, idx_v, bufs, sems):
    wid = lax.axis_index("s") * _NC + lax.axis_index("c")
    idx_row0 = wid * (_SPW * 2)
    seq0 = wid * _SPW

    idx_cp = pltpu.async_copy(
        idx_hbm.at[pl.ds(idx_row0, _SPW * 2)], idx_v, sems.at[6])
    pos_cp = pltpu.async_copy(pos_hbm, pos_v, sems.at[7])
    idx_cp.wait()

    def gather_start(i, slot):
        cps = []
        for h in range(2):
            cps.append(pltpu.async_copy(
                table_hbm.at[idx_v.at[2 * i + h]],
                bufs.at[slot].at[pl.ds(h * _IDX_MINOR, _IDX_MINOR)],
                sems.at[slot]))
        return cps

    def scatter_start(i, slot):
        return pltpu.async_copy(
            bufs.at[slot],
            out_hbm.at[pl.ds((seq0 + i) * _T, _T)],
            sems.at[_NBUF + slot])

    def fma(slot):
        def body(r, carry):
            for j in range(_D // 16):
                cs = pl.ds(j * 16, 16)
                bufs[slot, r, cs] = bufs[slot, r, cs] * _SCALE + pos_v[r, cs]
            return carry
        lax.fori_loop(0, _T, body, 0)

    pending_g = {0: gather_start(0, 0)}
    pending_s = {}
    for i in range(_SPW):
        slot = i % _NBUF
        if i + 1 < _SPW:
            nxt = (i + 1) % _NBUF
            if (i + 1) >= _NBUF:
                pending_s.pop(i + 1 - _NBUF).wait()
            pending_g[i + 1] = gather_start(i + 1, nxt)
        for cp in pending_g.pop(i):
            cp.wait()
        if i == 0:
            pos_cp.wait()
        fma(slot)
        pending_s[i] = scatter_start(i, slot)
    for i in sorted(pending_s):
        pending_s.pop(i).wait()


@functools.partial(
    pl.kernel,
    out_type=jax.ShapeDtypeStruct((_B * _T, _D), jnp.float32),
    mesh=plsc.VectorSubcoreMesh(core_axis_name="c", subcore_axis_name="s"),
    scratch_types=[
        pltpu.VMEM((_T, _D), jnp.float32),
        pltpu.VMEM((_SPW * 2, _IDX_MINOR), jnp.int32),
        pltpu.VMEM((_NBUF, _T, _D), jnp.float32),
        pltpu.SemaphoreType.DMA((8,)),
    ],
)
def _sc_embed(idx_hbm, table_hbm, pos_hbm, out_hbm, *rest):
    _sc_body(idx_hbm, table_hbm, pos_hbm, out_hbm, *rest)


def kernel(input, table):
    idx = input.reshape(_B * _T).astype(jnp.int32).reshape(-1, _IDX_MINOR)
    pos = jnp.asarray(_pos_table())
    out = _sc_embed(idx, table, pos)
    return out.reshape(_B, _T, _D)
